# baseline (speedup 1.0000x reference)
import functools

import jax
import jax.numpy as jnp
from jax import lax
from jax.experimental import pallas as pl
from jax.experimental.pallas import tpu as pltpu
from jax.experimental.pallas import tpu_sc as plsc

NC = 2
NS = 16
NW = NC * NS
L = 16

_HI = lax.Precision.DEFAULT


def _apsq_body(a_ref, p_ref, o_ref):
    @pl.when(pl.program_id(0) == 0)
    def _():
        o_ref[0, 0] = 0.0

    dd = a_ref[...] - p_ref[...]
    o_ref[0, 0] += jnp.sum(dd * dd)


def _apsq(a, p, bb=2048):
    B, d = a.shape
    return pl.pallas_call(
        _apsq_body,
        grid=(B // bb,),
        in_specs=[
            pl.BlockSpec((bb, d), lambda i: (i, 0)),
            pl.BlockSpec((bb, d), lambda i: (i, 0)),
        ],
        out_specs=pl.BlockSpec((1, 1), lambda i: (0, 0), memory_space=pltpu.SMEM),
        out_shape=jax.ShapeDtypeStruct((1, 1), jnp.float32),
    )(a, p)


def _w_body(bv, w_ref, sj_ref, relw_ref, c_ref, wj_ref):
    i = pl.program_id(0)

    @pl.when(i == 0)
    def _():
        relw_ref[0, 0] = 0.0
        wj_ref[...] = jnp.zeros_like(wj_ref)

    w = w_ref[...]
    rw = jnp.maximum(w, 0.0)
    relw_ref[0, 0] += jnp.sum(rw * rw)
    c_ref[0, :] = jnp.sum(w * w, axis=0)
    sj = sj_ref[0, :]
    iota = lax.broadcasted_iota(jnp.int32, (bv, sj.shape[0]), 0) + i * bv
    oh = (iota == sj[None, :]).astype(jnp.float32)
    wj_ref[...] += jax.lax.dot_general(
        w, oh, (((1,), (0,)), ((), ())), precision=_HI,
        preferred_element_type=jnp.float32)


def _w_pass(W, sj, bv=1024):
    d, V = W.shape
    M = sj.shape[1]
    return pl.pallas_call(
        functools.partial(_w_body, bv),
        grid=(V // bv,),
        in_specs=[
            pl.BlockSpec((d, bv), lambda i: (0, i)),
            pl.BlockSpec((1, M), lambda i: (0, 0)),
        ],
        out_specs=[
            pl.BlockSpec((1, 1), lambda i: (0, 0), memory_space=pltpu.SMEM),
            pl.BlockSpec((1, bv), lambda i: (0, i)),
            pl.BlockSpec((d, M), lambda i: (0, 0)),
        ],
        out_shape=[
            jax.ShapeDtypeStruct((1, 1), jnp.float32),
            jax.ShapeDtypeStruct((1, V), jnp.float32),
            jax.ShapeDtypeStruct((d, M), jnp.float32),
        ],
    )(W, sj)


def _g_body(wj_ref, w_ref, g_ref):
    g_ref[...] = jax.lax.dot_general(
        wj_ref[...], w_ref[...], (((0,), (0,)), ((), ())), precision=_HI,
        preferred_element_type=jnp.float32)


def _g_pass(wj, W, bv=1024):
    d, V = W.shape
    M = wj.shape[1]
    return pl.pallas_call(
        _g_body,
        grid=(V // bv,),
        in_specs=[
            pl.BlockSpec((d, M), lambda i: (0, 0)),
            pl.BlockSpec((d, bv), lambda i: (0, i)),
        ],
        out_specs=pl.BlockSpec((M, bv), lambda i: (0, i)),
        out_shape=jax.ShapeDtypeStruct((M, V), jnp.float32),
    )(wj, W)


def _e_body(e_ref, ej_ref, ri_ref, rele_ref, desq_ref):
    e = e_ref[...]
    re = jnp.maximum(e, 0.0)
    rele_ref[0, 0] = jnp.sum(re * re)
    V = e.shape[0]
    ej = ej_ref[0, :]
    iota = lax.broadcasted_iota(jnp.int32, (V, ej.shape[0]), 0)
    oh = (iota == ej[None, :]).astype(jnp.float32)
    ej_rows = jax.lax.dot_general(
        oh, e, (((0,), (0,)), ((), ())), precision=_HI,
        preferred_element_type=jnp.float32)
    ri = ri_ref[0]
    oh_r = (lax.broadcasted_iota(jnp.int32, (V, 1), 0) == ri).astype(jnp.float32)
    ei_row = jax.lax.dot_general(
        oh_r, e, (((0,), (0,)), ((), ())), precision=_HI,
        preferred_element_type=jnp.float32)
    dd = ej_rows - ei_row
    desq_ref[0, 0] = jnp.sum(dd * dd)


def _e_pass(E, ej, ri1):
    V, d = E.shape
    M2 = ej.shape[1]
    return pl.pallas_call(
        _e_body,
        in_specs=[
            pl.BlockSpec((V, d), lambda: (0, 0)),
            pl.BlockSpec((1, M2), lambda: (0, 0)),
            pl.BlockSpec(memory_space=pltpu.SMEM),
        ],
        out_specs=[
            pl.BlockSpec(memory_space=pltpu.SMEM),
            pl.BlockSpec(memory_space=pltpu.SMEM),
        ],
        out_shape=[
            jax.ShapeDtypeStruct((1, 1), jnp.float32),
            jax.ShapeDtypeStruct((1, 1), jnp.float32),
        ],
    )(E, ej, ri1)


def _make_sc_sw(V, M, K, M2, JS):
    mesh = plsc.VectorSubcoreMesh(
        core_axis_name="c", subcore_axis_name="s",
        num_cores=NC, num_subcores=NS)

    @functools.partial(
        pl.kernel,
        compiler_params=pltpu.CompilerParams(needs_layout_passes=False),
        out_type=[
            jax.ShapeDtypeStruct((M,), jnp.float32),
            jax.ShapeDtypeStruct((L,), jnp.float32),
        ],
        mesh=mesh,
        scratch_types=[
            pltpu.VMEM((8, V), jnp.float32),
            pltpu.VMEM((8, V), jnp.float32),
            pltpu.VMEM((K, JS), jnp.int32),
            pltpu.VMEM((JS,), jnp.int32),
            pltpu.VMEM((8,), jnp.int32),
            pltpu.VMEM((M2,), jnp.int32),
            pltpu.VMEM((L,), jnp.float32),
            pltpu.SemaphoreType.DMA,
            pltpu.SemaphoreType.DMA,
        ],
    )
    def sc_sw(sw_hbm, se_hbm, wiT_hbm, sj_hbm, ri_hbm, ej_hbm,
              swsum_hbm, sesum_hbm,
              bufa, bufb, wiT_v, sj_v, ri_v, ej_v, stage, sema, semb):
        wid = lax.axis_index("s") * NC + lax.axis_index("c")
        base = wid * JS
        pltpu.sync_copy(sj_hbm.at[pl.ds(base, JS)], sj_v)
        cp_sw0 = pltpu.async_copy(sw_hbm.at[sj_v.at[pl.ds(0, 8)]], bufa, sema)
        cp_sw1 = pltpu.async_copy(sw_hbm.at[sj_v.at[pl.ds(8, 8)]], bufb, semb)
        pltpu.sync_copy(wiT_hbm.at[wid], wiT_v)

        lanes = lax.iota(jnp.int32, L)
        rows = lanes & 7
        lo = lanes < 8
        hi = jnp.logical_not(lo)
        zerov = jnp.zeros((L,), jnp.float32)

        def gather_sum(b_ref):
            acc = zerov
            for k in range(K):
                acc = acc + plsc.load_gather(b_ref, [rows, wiT_v[k]])
            return acc

        cp_sw0.wait()
        sw_lo = gather_sum(bufa)
        cp_sw1.wait()
        sw_hi = gather_sum(bufb)
        stage[...] = jnp.where(lo, sw_lo, zerov) + jnp.where(hi, sw_hi, zerov)
        pltpu.sync_copy(stage, swsum_hbm.at[pl.ds(base, JS)])

        @pl.when(wid == 0)
        def _():
            pltpu.sync_copy(ri_hbm, ri_v)
            pltpu.sync_copy(ej_hbm, ej_v)
            pltpu.async_copy(se_hbm.at[ri_v], bufa, sema).wait()
            zeros = jnp.zeros((L,), jnp.int32)
            se_acc = zerov
            for t in range(M2 // L):
                idx = ej_v[pl.ds(t * L, L)]
                se_acc = se_acc + plsc.load_gather(bufa, [zeros, idx])
            s = jnp.sum(se_acc)
            stage[...] = jnp.full((L,), s, jnp.float32)
            pltpu.sync_copy(stage, sesum_hbm)

    return sc_sw


def _make_sc_g(V, M, K, JS):
    mesh = plsc.VectorSubcoreMesh(
        core_axis_name="c", subcore_axis_name="s",
        num_cores=NC, num_subcores=NS)

    @functools.partial(
        pl.kernel,
        compiler_params=pltpu.CompilerParams(needs_layout_passes=False),
        out_type=[
            jax.ShapeDtypeStruct((M,), jnp.float32),
            jax.ShapeDtypeStruct((M,), jnp.float32),
            jax.ShapeDtypeStruct((M,), jnp.float32),
        ],
        mesh=mesh,
        scratch_types=[
            pltpu.VMEM((8, V), jnp.float32),
            pltpu.VMEM((8, V), jnp.float32),
            pltpu.VMEM((K, JS), jnp.int32),
            pltpu.VMEM((JS,), jnp.int32),
            pltpu.VMEM((V,), jnp.float32),
            pltpu.VMEM((L,), jnp.float32),
            pltpu.SemaphoreType.DMA,
            pltpu.SemaphoreType.DMA,
        ],
    )
    def sc_g(g_hbm, c_hbm, wiT_hbm, sj_hbm,
             gsum_hbm, csum_hbm, cj_hbm,
             bufa, bufb, wiT_v, sj_v, c_v, stage, sema, semb):
        wid = lax.axis_index("s") * NC + lax.axis_index("c")
        base = wid * JS
        cp_g0 = pltpu.async_copy(g_hbm.at[pl.ds(base, 8), :], bufa, sema)
        cp_g1 = pltpu.async_copy(g_hbm.at[pl.ds(base + 8, 8), :], bufb, semb)
        pltpu.sync_copy(sj_hbm.at[pl.ds(base, JS)], sj_v)
        pltpu.sync_copy(wiT_hbm.at[wid], wiT_v)
        pltpu.sync_copy(c_hbm, c_v)

        lanes = lax.iota(jnp.int32, L)
        rows = lanes & 7
        lo = lanes < 8
        hi = jnp.logical_not(lo)
        zerov = jnp.zeros((L,), jnp.float32)

        acc = zerov
        for k in range(K):
            acc = acc + plsc.load_gather(c_v, [wiT_v[k]])
        stage[...] = acc
        pltpu.sync_copy(stage, csum_hbm.at[pl.ds(base, JS)])
        stage[...] = plsc.load_gather(c_v, [sj_v[...]])
        pltpu.sync_copy(stage, cj_hbm.at[pl.ds(base, JS)])

        def gather_sum(b_ref):
            acc = zerov
            for k in range(K):
                acc = acc + plsc.load_gather(b_ref, [rows, wiT_v[k]])
            return acc

        cp_g0.wait()
        g_lo = gather_sum(bufa)
        cp_g1.wait()
        g_hi = gather_sum(bufb)
        stage[...] = jnp.where(lo, g_lo, zerov) + jnp.where(hi, g_hi, zerov)
        pltpu.sync_copy(stage, gsum_hbm.at[pl.ds(base, JS)])

    return sc_g


def _combine_body(K, apsq_ref, relw_ref, rele_ref, desq_ref, sesum_ref,
                  lamb_ref, swsum_ref, gsum_ref, csum_ref, cj_ref, out_ref):
    t = K * cj_ref[...] - 2.0 * gsum_ref[...] + csum_ref[...]
    t = jnp.maximum(t, 0.0)
    sim = jnp.sum(jnp.sqrt(t) * swsum_ref[...])
    diff_e = jnp.sqrt(jnp.maximum(desq_ref[0, 0], 0.0))
    sim = sim + diff_e * sesum_ref[0, 0]
    out_ref[0, 0] = (jnp.sqrt(apsq_ref[0, 0])
                     + lamb_ref[0, 0] * (jnp.sqrt(relw_ref[0, 0])
                                         + jnp.sqrt(rele_ref[0, 0]))
                     + sim)


def _combine(K, apsq, relw, rele, desq, sesum, lamb, swsum, gsum, csum, cj):
    M = swsum.shape[1]
    smem = pl.BlockSpec(memory_space=pltpu.SMEM)
    vm = pl.BlockSpec((1, M), lambda: (0, 0))
    return pl.pallas_call(
        functools.partial(_combine_body, float(K)),
        in_specs=[smem, smem, smem, smem, smem, smem, vm, vm, vm, vm],
        out_specs=pl.BlockSpec(memory_space=pltpu.SMEM),
        out_shape=jax.ShapeDtypeStruct((1, 1), jnp.float32),
    )(apsq, relw, rele, desq, sesum, lamb, swsum, gsum, csum, cj)


def kernel(actual, prediction, W, E, Sw, Se, lamb, row_ind,
           word_i_indices, entity_j_indices, sample_j_indices):
    d, V = W.shape
    M, K = word_i_indices.shape
    M2 = entity_j_indices.shape[0]
    JS = M // NW

    sj = jnp.asarray(sample_j_indices, jnp.int32)
    ej = jnp.asarray(entity_j_indices, jnp.int32)
    ri = jnp.asarray(row_ind, jnp.int32)
    ri1 = jnp.reshape(ri, (1,))
    wi = jnp.asarray(word_i_indices, jnp.int32)
    wiT = jnp.transpose(wi.reshape(NW, JS, K), (0, 2, 1))
    ri8 = jnp.full((8,), ri, jnp.int32)

    apsq = _apsq(actual, prediction)
    relw, c, wj = _w_pass(W, jnp.reshape(sj, (1, M)))
    g = _g_pass(wj, W)
    rele, desq = _e_pass(E, jnp.reshape(ej, (1, M2)), ri1)

    swsum, sesum = _make_sc_sw(V, M, K, M2, JS)(Sw, Se, wiT, sj, ri8, ej)
    gsum, csum, cjv = _make_sc_g(V, M, K, JS)(
        g, jnp.reshape(c, (V,)), wiT, sj)

    out = _combine(
        K, apsq, relw, rele, desq,
        jnp.reshape(sesum, (1, L)), jnp.reshape(lamb, (1, 1)),
        jnp.reshape(swsum, (1, M)), jnp.reshape(gsum, (1, M)),
        jnp.reshape(csum, (1, M)), jnp.reshape(cjv, (1, M)))
    return out[0, 0]


# speedup vs baseline: 1.0525x; 1.0525x over previous
import functools

import jax
import jax.numpy as jnp
from jax import lax
from jax.experimental import pallas as pl
from jax.experimental.pallas import tpu as pltpu
from jax.experimental.pallas import tpu_sc as plsc

NC = 2
NS = 16
NW = NC * NS
L = 16


def _wg_body(bv, nb, w_ref, sj_ref, relw_ref, c_ref, g_ref, wj_ref):
    i = pl.program_id(0)

    @pl.when(i == 0)
    def _():
        relw_ref[0, 0] = 0.0
        wj_ref[...] = jnp.zeros_like(wj_ref)

    @pl.when(i < nb)
    def _():
        w = w_ref[...]
        rw = jnp.maximum(w, 0.0)
        relw_ref[0, 0] += jnp.sum(rw * rw)
        c_ref[0, :] = jnp.sum(w * w, axis=0)
        sj = sj_ref[0, :]
        iota = lax.broadcasted_iota(jnp.int32, (bv, sj.shape[0]), 0) + i * bv
        oh = (iota == sj[None, :]).astype(jnp.float32)
        wj_ref[...] += jax.lax.dot_general(
            w, oh, (((1,), (0,)), ((), ())),
            preferred_element_type=jnp.float32)

    @pl.when(i >= nb)
    def _():
        g_ref[...] = jax.lax.dot_general(
            wj_ref[...], w_ref[...], (((0,), (0,)), ((), ())),
            preferred_element_type=jnp.float32)


def _wg_pass(W, sj, bv=1024):
    d, V = W.shape
    M = sj.shape[1]
    nb = V // bv
    return pl.pallas_call(
        functools.partial(_wg_body, bv, nb),
        grid=(2 * nb,),
        in_specs=[
            pl.BlockSpec((d, bv), lambda i: (0, jnp.where(i < nb, i, i - nb))),
            pl.BlockSpec((1, M), lambda i: (0, 0)),
        ],
        out_specs=[
            pl.BlockSpec((1, 1), lambda i: (0, 0), memory_space=pltpu.SMEM),
            pl.BlockSpec((1, bv), lambda i: (0, jnp.where(i < nb, i, nb - 1))),
            pl.BlockSpec((M, bv), lambda i: (0, jnp.where(i < nb, 0, i - nb))),
        ],
        out_shape=[
            jax.ShapeDtypeStruct((1, 1), jnp.float32),
            jax.ShapeDtypeStruct((1, V), jnp.float32),
            jax.ShapeDtypeStruct((M, V), jnp.float32),
        ],
        scratch_shapes=[pltpu.VMEM((d, M), jnp.float32)],
    )(W, sj)


def _ape_body(bb, nb, ne, M2, a_ref, p_ref, e_ref, ej_ref, ri_ref,
              apsq_ref, rele_ref, desq_ref, ejr_ref, ei_ref):
    i = pl.program_id(0)

    @pl.when(i == 0)
    def _():
        apsq_ref[0, 0] = 0.0
        rele_ref[0, 0] = 0.0
        ejr_ref[...] = jnp.zeros_like(ejr_ref)
        ei_ref[...] = jnp.zeros_like(ei_ref)

    @pl.when(i < nb)
    def _():
        dd = a_ref[...] - p_ref[...]
        apsq_ref[0, 0] += jnp.sum(dd * dd)

    @pl.when(i >= nb)
    def _():
        e = e_ref[...]
        re = jnp.maximum(e, 0.0)
        rele_ref[0, 0] += jnp.sum(re * re)
        base = (i - nb) * bb
        ej = ej_ref[0, :]
        iota = lax.broadcasted_iota(jnp.int32, (bb, M2), 0) + base
        oh = (iota == ej[None, :]).astype(jnp.float32)
        ejr_ref[...] += jax.lax.dot_general(
            oh, e, (((0,), (0,)), ((), ())),
            preferred_element_type=jnp.float32)
        ri = ri_ref[0]
        oh_r = (lax.broadcasted_iota(jnp.int32, (bb, 1), 0) + base ==
                ri).astype(jnp.float32)
        ei_ref[...] += jax.lax.dot_general(
            oh_r, e, (((0,), (0,)), ((), ())),
            preferred_element_type=jnp.float32)

    @pl.when(i == nb + ne - 1)
    def _():
        dd = ejr_ref[...] - ei_ref[...]
        desq_ref[0, 0] = jnp.sum(dd * dd)


def _ape_pass(a, p, E, ej, ri1, bb=2048):
    B, d = a.shape
    V, _ = E.shape
    M2 = ej.shape[1]
    nb = B // bb
    ne = V // bb
    smem = pl.BlockSpec(memory_space=pltpu.SMEM)
    return pl.pallas_call(
        functools.partial(_ape_body, bb, nb, ne, M2),
        grid=(nb + ne,),
        in_specs=[
            pl.BlockSpec((bb, d), lambda i: (jnp.where(i < nb, i, nb - 1), 0)),
            pl.BlockSpec((bb, d), lambda i: (jnp.where(i < nb, i, nb - 1), 0)),
            pl.BlockSpec((bb, d), lambda i: (jnp.where(i < nb, 0, i - nb), 0)),
            pl.BlockSpec((1, M2), lambda i: (0, 0)),
            smem,
        ],
        out_specs=[smem, smem, smem],
        out_shape=[
            jax.ShapeDtypeStruct((1, 1), jnp.float32),
            jax.ShapeDtypeStruct((1, 1), jnp.float32),
            jax.ShapeDtypeStruct((1, 1), jnp.float32),
        ],
        scratch_shapes=[
            pltpu.VMEM((M2, d), jnp.float32),
            pltpu.VMEM((1, d), jnp.float32),
        ],
    )(a, p, E, ej, ri1)


def _make_sc_gather(V, M, K, M2, JS):
    mesh = plsc.VectorSubcoreMesh(
        core_axis_name="c", subcore_axis_name="s",
        num_cores=NC, num_subcores=NS)

    @functools.partial(
        pl.kernel,
        compiler_params=pltpu.CompilerParams(needs_layout_passes=False),
        out_type=[
            jax.ShapeDtypeStruct((M,), jnp.float32),
            jax.ShapeDtypeStruct((M,), jnp.float32),
            jax.ShapeDtypeStruct((M,), jnp.float32),
            jax.ShapeDtypeStruct((M,), jnp.float32),
            jax.ShapeDtypeStruct((L,), jnp.float32),
        ],
        mesh=mesh,
        scratch_types=[
            pltpu.VMEM((8, V), jnp.float32),
            pltpu.VMEM((8, V), jnp.float32),
            pltpu.VMEM((8, V), jnp.float32),
            pltpu.VMEM((K, JS), jnp.int32),
            pltpu.VMEM((JS,), jnp.int32),
            pltpu.VMEM((V,), jnp.float32),
            pltpu.VMEM((8,), jnp.int32),
            pltpu.VMEM((M2,), jnp.int32),
            pltpu.VMEM((L,), jnp.float32),
            pltpu.SemaphoreType.DMA,
            pltpu.SemaphoreType.DMA,
            pltpu.SemaphoreType.DMA,
        ],
    )
    def sc_gather(sw_hbm, se_hbm, g_hbm, c_hbm,
                  wiT_hbm, sj_hbm, ri_hbm, ej_hbm,
                  swsum_hbm, gsum_hbm, csum_hbm, cj_hbm, sesum_hbm,
                  bufa, bufb, bufc, wiT_v, sj_v, c_v, ri_v, ej_v, stage,
                  sema, semb, semc):
        wid = lax.axis_index("s") * NC + lax.axis_index("c")
        base = wid * JS
        pltpu.sync_copy(sj_hbm.at[pl.ds(base, JS)], sj_v)
        pltpu.sync_copy(wiT_hbm.at[wid], wiT_v)

        cp_sw0 = pltpu.async_copy(sw_hbm.at[sj_v.at[pl.ds(0, 8)]], bufa, sema)
        cp_sw1 = pltpu.async_copy(sw_hbm.at[sj_v.at[pl.ds(8, 8)]], bufb, semb)
        cp_g0 = pltpu.async_copy(g_hbm.at[pl.ds(base, 8), :], bufc, semc)
        pltpu.sync_copy(c_hbm, c_v)

        lanes = lax.iota(jnp.int32, L)
        rows = lanes & 7
        lo = lanes < 8
        hi = jnp.logical_not(lo)
        zerov = jnp.zeros((L,), jnp.float32)

        acc = zerov
        for k in range(K):
            acc = acc + plsc.load_gather(c_v, [wiT_v[k]])
        stage[...] = acc
        pltpu.sync_copy(stage, csum_hbm.at[pl.ds(base, JS)])
        stage[...] = plsc.load_gather(c_v, [sj_v[...]])
        pltpu.sync_copy(stage, cj_hbm.at[pl.ds(base, JS)])

        def gather_sum(b_ref):
            acc = zerov
            for k in range(K):
                acc = acc + plsc.load_gather(b_ref, [rows, wiT_v[k]])
            return acc

        cp_sw0.wait()
        sw_lo = gather_sum(bufa)
        cp_g1 = pltpu.async_copy(g_hbm.at[pl.ds(base + 8, 8), :], bufa, sema)
        cp_sw1.wait()
        sw_hi = gather_sum(bufb)
        stage[...] = jnp.where(lo, sw_lo, zerov) + jnp.where(hi, sw_hi, zerov)
        pltpu.sync_copy(stage, swsum_hbm.at[pl.ds(base, JS)])

        @pl.when(wid == 0)
        def _():
            pltpu.sync_copy(ri_hbm, ri_v)
            pltpu.sync_copy(ej_hbm, ej_v)
            pltpu.async_copy(se_hbm.at[ri_v], bufb, semb)

        cp_g0.wait()
        g_lo = gather_sum(bufc)
        cp_g1.wait()
        g_hi = gather_sum(bufa)
        stage[...] = jnp.where(lo, g_lo, zerov) + jnp.where(hi, g_hi, zerov)
        pltpu.sync_copy(stage, gsum_hbm.at[pl.ds(base, JS)])

        @pl.when(wid == 0)
        def _():
            pltpu.make_async_copy(se_hbm.at[ri_v], bufb, semb).wait()
            zeros = jnp.zeros((L,), jnp.int32)
            se_acc = zerov
            for t in range(M2 // L):
                idx = ej_v[pl.ds(t * L, L)]
                se_acc = se_acc + plsc.load_gather(bufb, [zeros, idx])
            s = jnp.sum(se_acc)
            stage[...] = jnp.full((L,), s, jnp.float32)
            pltpu.sync_copy(stage, sesum_hbm)

    return sc_gather


def _combine_body(K, apsq_ref, relw_ref, rele_ref, desq_ref, sesum_ref,
                  lamb_ref, swsum_ref, gsum_ref, csum_ref, cj_ref, out_ref):
    t = K * cj_ref[...] - 2.0 * gsum_ref[...] + csum_ref[...]
    t = jnp.maximum(t, 0.0)
    sim = jnp.sum(jnp.sqrt(t) * swsum_ref[...])
    diff_e = jnp.sqrt(jnp.maximum(desq_ref[0, 0], 0.0))
    sim = sim + diff_e * sesum_ref[0, 0]
    out_ref[0, 0] = (jnp.sqrt(apsq_ref[0, 0])
                     + lamb_ref[0, 0] * (jnp.sqrt(relw_ref[0, 0])
                                         + jnp.sqrt(rele_ref[0, 0]))
                     + sim)


def _combine(K, apsq, relw, rele, desq, sesum, lamb, swsum, gsum, csum, cj):
    M = swsum.shape[1]
    smem = pl.BlockSpec(memory_space=pltpu.SMEM)
    vm = pl.BlockSpec((1, M), lambda: (0, 0))
    return pl.pallas_call(
        functools.partial(_combine_body, float(K)),
        in_specs=[smem, smem, smem, smem, smem, smem, vm, vm, vm, vm],
        out_specs=pl.BlockSpec(memory_space=pltpu.SMEM),
        out_shape=jax.ShapeDtypeStruct((1, 1), jnp.float32),
    )(apsq, relw, rele, desq, sesum, lamb, swsum, gsum, csum, cj)


def kernel(actual, prediction, W, E, Sw, Se, lamb, row_ind,
           word_i_indices, entity_j_indices, sample_j_indices):
    d, V = W.shape
    M, K = word_i_indices.shape
    M2 = entity_j_indices.shape[0]
    JS = M // NW

    sj = jnp.asarray(sample_j_indices, jnp.int32)
    ej = jnp.asarray(entity_j_indices, jnp.int32)
    ri = jnp.asarray(row_ind, jnp.int32)
    ri1 = jnp.reshape(ri, (1,))
    wi = jnp.asarray(word_i_indices, jnp.int32)
    wiT = jnp.transpose(wi.reshape(NW, JS, K), (0, 2, 1))
    ri8 = jnp.full((8,), ri, jnp.int32)

    relw, c, g = _wg_pass(W, jnp.reshape(sj, (1, M)))
    apsq, rele, desq = _ape_pass(
        actual, prediction, E, jnp.reshape(ej, (1, M2)), ri1)

    swsum, gsum, csum, cjv, sesum = _make_sc_gather(V, M, K, M2, JS)(
        Sw, Se, g, jnp.reshape(c, (V,)), wiT, sj, ri8, ej)

    out = _combine(
        K, apsq, relw, rele, desq,
        jnp.reshape(sesum, (1, L)), jnp.reshape(lamb, (1, 1)),
        jnp.reshape(swsum, (1, M)), jnp.reshape(gsum, (1, M)),
        jnp.reshape(csum, (1, M)), jnp.reshape(cjv, (1, M)))
    return out[0, 0]
